# baseline (speedup 1.0000x reference)
import jax
import jax.numpy as jnp
from jax.experimental import pallas as pl

_K = 256


def _topk_mask_kernel(x_ref, o_ref):
    x = x_ref[...]
    b = jax.lax.bitcast_convert_type(x, jnp.int32)
    key = b ^ ((b >> 31) & jnp.int32(0x7FFFFFFF))
    key_hi = (key >> 16).astype(jnp.int16)

    cnt = jnp.sum((key_hi >= 0).astype(jnp.int16), axis=1, keepdims=True)
    t_hi = jnp.where(cnt >= _K, jnp.int32(0), jnp.int32(-32768))

    def body_hi(i, t):
        cand = t | (jnp.int32(1) << (14 - i))
        cand16 = cand.astype(jnp.int16)
        c = jnp.sum((key_hi >= cand16).astype(jnp.int16), axis=1, keepdims=True)
        return jnp.where(c >= _K, cand, t)

    t_hi = jax.lax.fori_loop(0, 15, body_hi, t_hi)
    t_hi16 = t_hi.astype(jnp.int16)

    lo_s = ((key & jnp.int32(0xFFFF)) - 32768).astype(jnp.int16)
    gt = key_hi > t_hi16
    eq = key_hi == t_hi16
    slo = jnp.where(gt, jnp.int16(32767),
                    jnp.where(eq, lo_s, jnp.int16(-32768)))

    t_lo = jnp.full_like(t_hi, jnp.int32(-32768))

    def body_lo(i, t):
        cand = t + (jnp.int32(1) << (15 - i))
        cand16 = cand.astype(jnp.int16)
        c = jnp.sum((slo >= cand16).astype(jnp.int16), axis=1, keepdims=True)
        return jnp.where(c >= _K, cand, t)

    t_lo = jax.lax.fori_loop(0, 16, body_lo, t_lo)

    t_full = (t_hi << 16) | (t_lo + 32768)
    thr_bits = t_full ^ ((t_full >> 31) & jnp.int32(0x7FFFFFFF))
    thr = jax.lax.bitcast_convert_type(thr_bits, jnp.float32)
    o_ref[...] = jnp.where(x >= thr, x, jnp.zeros_like(x))


@jax.jit
def kernel(x):
    return pl.pallas_call(
        _topk_mask_kernel,
        out_shape=jax.ShapeDtypeStruct(x.shape, x.dtype),
    )(x)


# speedup vs baseline: 1.6916x; 1.6916x over previous
import jax
import jax.numpy as jnp
from jax.experimental import pallas as pl

_K = 256


def _search(key):
    cnt = jnp.sum((key >= 0).astype(jnp.int32), axis=1, keepdims=True)
    t = jnp.where(cnt >= _K, jnp.int32(0), jnp.int32(-2147483648))
    for bit in range(30, -1, -1):
        cand = t | (jnp.int32(1) << bit)
        cnt = jnp.sum((key >= cand).astype(jnp.int32), axis=1, keepdims=True)
        t = jnp.where(cnt >= _K, cand, t)
    return t


def _topk_mask_kernel(x_ref, o_ref):
    x = x_ref[...]
    b = jax.lax.bitcast_convert_type(x, jnp.int32)
    key = b ^ ((b >> 31) & jnp.int32(0x7FFFFFFF))

    half = x.shape[0] // 2
    t0 = _search(key[:half])
    t1 = _search(key[half:])
    t = jnp.concatenate([t0, t1], axis=0)

    thr_bits = t ^ ((t >> 31) & jnp.int32(0x7FFFFFFF))
    thr = jax.lax.bitcast_convert_type(thr_bits, jnp.float32)
    o_ref[...] = jnp.where(x >= thr, x, jnp.zeros_like(x))


@jax.jit
def kernel(x):
    return pl.pallas_call(
        _topk_mask_kernel,
        out_shape=jax.ShapeDtypeStruct(x.shape, x.dtype),
    )(x)
